# Initial kernel scaffold; baseline (speedup 1.0000x reference)
import jax
import jax.numpy as jnp
from jax.experimental import pallas as pl


def kernel(x, edge_index, edge_attr, u, embed, params):
    raise NotImplementedError("write your pallas kernel here")


# speedup vs baseline: 2.4150x; 2.4150x over previous
import functools

import jax
import jax.numpy as jnp
from jax import lax
from jax.experimental import pallas as pl
from jax.experimental.pallas import tpu as pltpu
from jax.experimental.pallas import tpu_sc as plsc

F32 = jnp.float32

_NC = 2
_NS = 16
_NW = _NC * _NS
_CH = 80

_RB = 3200


def _mlp_ln(h, W1, b1, W2, b2, g, b):
    h = jnp.maximum(jnp.dot(h, W1, preferred_element_type=F32) + b1, 0.0)
    h = jnp.maximum(jnp.dot(h, W2, preferred_element_type=F32) + b2, 0.0)
    m = jnp.mean(h, axis=-1, keepdims=True)
    v = jnp.mean((h - m) * (h - m), axis=-1, keepdims=True)
    return (h - m) * lax.rsqrt(v + 1e-5) * g + b


def _row(a):
    return a.reshape(1, -1)


def _mlp_args(p):
    (W1, b1), (W2, b2) = p["lins"]
    return (W1, _row(b1), W2, _row(b2), _row(p["g"]), _row(p["b"]))


def _k_node_enc(x_r, emb_r, u_r,
                nW1, nb1, nW2, nb2, ng, nb,
                gW1, gb1, gW2, gb2, gg, gb,
                wra, wca, wua, b1e,
                xv_o, ug_o, a_o, qu_o):
    xv = _mlp_ln(x_r[...] + emb_r[...], nW1[...], nb1[...], nW2[...], nb2[...],
                 ng[...], nb[...])
    ug = _mlp_ln(u_r[...], gW1[...], gb1[...], gW2[...], gb2[...], gg[...], gb[...])
    xv_o[...] = xv
    ug_o[...] = ug
    a_o[...] = jnp.concatenate(
        [jnp.dot(xv, wra[...], preferred_element_type=F32),
         jnp.dot(xv, wca[...], preferred_element_type=F32)], axis=1)
    qu_o[...] = jnp.dot(ug, wua[...], preferred_element_type=F32) + b1e[...]


def node_enc(x, embed, u, p, wra, wca, wua, b1e):
    N = x.shape[0]
    out_shape = (
        jax.ShapeDtypeStruct((N, 64), F32),
        jax.ShapeDtypeStruct((1, 32), F32),
        jax.ShapeDtypeStruct((N, 128), F32),
        jax.ShapeDtypeStruct((1, 64), F32),
    )
    return pl.pallas_call(_k_node_enc, out_shape=out_shape)(
        x, embed, u, *_mlp_args(p["enc_node"]), *_mlp_args(p["enc_glob"]),
        wra, wca, wua, b1e)


def _k_edge_enc(ea_r, W1, b1, W2, b2, g, b, we, qu, e_o, q_o):
    e = _mlp_ln(ea_r[...], W1[...], b1[...], W2[...], b2[...], g[...], b[...])
    e_o[...] = e
    q_o[...] = jnp.dot(e, we[...], preferred_element_type=F32) + qu[...]


def edge_enc(edge_attr, p, we, qu):
    E = edge_attr.shape[0]
    grid = E // _RB
    full = lambda a: pl.BlockSpec(a.shape, lambda i: (0,) * a.ndim)
    args = (*_mlp_args(p["enc_edge"]), we, qu)
    return pl.pallas_call(
        _k_edge_enc,
        grid=(grid,),
        in_specs=[pl.BlockSpec((_RB, 16), lambda i: (i, 0))] + [full(a) for a in args],
        out_specs=(pl.BlockSpec((_RB, 64), lambda i: (i, 0)),
                   pl.BlockSpec((_RB, 64), lambda i: (i, 0))),
        out_shape=(jax.ShapeDtypeStruct((E, 64), F32),
                   jax.ShapeDtypeStruct((E, 64), F32)),
    )(edge_attr, *args)


def _k_edge_q2(e_r, le_r, we, wl, qu, q_o):
    q_o[...] = (jnp.dot(e_r[...], we[...], preferred_element_type=F32)
                + jnp.dot(le_r[...][:, :64], wl[...], preferred_element_type=F32)
                + qu[...])


def edge_q2(e, le, we, wl, qu):
    E = e.shape[0]
    grid = E // _RB
    full = lambda a: pl.BlockSpec(a.shape, lambda i: (0,) * a.ndim)
    eb = pl.BlockSpec((_RB, 64), lambda i: (i, 0))
    eb2 = pl.BlockSpec((_RB, 128), lambda i: (i, 0))
    return pl.pallas_call(
        _k_edge_q2,
        grid=(grid,),
        in_specs=[eb, eb2, full(we), full(wl), full(qu)],
        out_specs=eb,
        out_shape=jax.ShapeDtypeStruct((E, 64), F32),
    )(e, le, we, wl, qu)


def _k_edge_b(h_r, W2, b2, g, b, en_o, es_o):
    i = pl.program_id(0)
    h = jnp.maximum(jnp.dot(h_r[...], W2[...], preferred_element_type=F32) + b2[...], 0.0)
    m = jnp.mean(h, axis=-1, keepdims=True)
    v = jnp.mean((h - m) * (h - m), axis=-1, keepdims=True)
    en = (h - m) * lax.rsqrt(v + 1e-5) * g[...] + b[...]
    en_o[...] = jnp.concatenate([en, jnp.zeros_like(en)], axis=1)

    @pl.when(i == 0)
    def _():
        es_o[...] = jnp.zeros_like(es_o)

    es_o[...] += jnp.sum(en, axis=0, keepdims=True)


def edge_b(H, p):
    E = H.shape[0]
    grid = E // _RB
    (_, _), (W2, b2) = p["lins"]
    args = (W2, _row(b2), _row(p["g"]), _row(p["b"]))
    full = lambda a: pl.BlockSpec(a.shape, lambda i: (0,) * a.ndim)
    eb = pl.BlockSpec((_RB, 64), lambda i: (i, 0))
    eb2 = pl.BlockSpec((_RB, 128), lambda i: (i, 0))
    return pl.pallas_call(
        _k_edge_b,
        grid=(grid,),
        in_specs=[eb] + [full(a) for a in args],
        out_specs=(eb2, pl.BlockSpec((1, 64), lambda i: (0, 0))),
        out_shape=(jax.ShapeDtypeStruct((E, 128), F32),
                   jax.ShapeDtypeStruct((1, 64), F32)),
    )(H, *args)


def _k_edge_b_dec(h_r, W2, b2, g, b, dW1, db1, dW2, db2, dg, db, wo, bo,
                  en_o, oe_o, es_o):
    i = pl.program_id(0)
    h = jnp.maximum(jnp.dot(h_r[...], W2[...], preferred_element_type=F32) + b2[...], 0.0)
    m = jnp.mean(h, axis=-1, keepdims=True)
    v = jnp.mean((h - m) * (h - m), axis=-1, keepdims=True)
    en = (h - m) * lax.rsqrt(v + 1e-5) * g[...] + b[...]
    en_o[...] = jnp.concatenate([en, jnp.zeros_like(en)], axis=1)
    d = _mlp_ln(en, dW1[...], db1[...], dW2[...], db2[...], dg[...], db[...])
    oe_o[...] = jnp.dot(d, wo[...], preferred_element_type=F32) + bo[...]

    @pl.when(i == 0)
    def _():
        es_o[...] = jnp.zeros_like(es_o)

    es_o[...] += jnp.sum(en, axis=0, keepdims=True)


def edge_b_dec(H, p, pdec, wo, bo):
    E = H.shape[0]
    grid = E // _RB
    (_, _), (W2, b2) = p["lins"]
    args = (W2, _row(b2), _row(p["g"]), _row(p["b"]), *_mlp_args(pdec), wo, _row(bo))
    full = lambda a: pl.BlockSpec(a.shape, lambda i: (0,) * a.ndim)
    eb = pl.BlockSpec((_RB, 64), lambda i: (i, 0))
    eb2 = pl.BlockSpec((_RB, 128), lambda i: (i, 0))
    return pl.pallas_call(
        _k_edge_b_dec,
        grid=(grid,),
        in_specs=[eb] + [full(a) for a in args],
        out_specs=(eb2, pl.BlockSpec((_RB, 16), lambda i: (i, 0)),
                   pl.BlockSpec((1, 64), lambda i: (0, 0))),
        out_shape=(jax.ShapeDtypeStruct((E, 128), F32),
                   jax.ShapeDtypeStruct((E, 16), F32),
                   jax.ShapeDtypeStruct((1, 64), F32)),
    )(H, *args)


def _k_node_core(xv_r, lv_r, agg_r, uu_r, esum_r, ug_r, lu_r,
                 wxa, wxl, wagg,
                 nW2, nb2, ng, nb,
                 guu_r, gwx, gwe, gW2, gb2, gg, gb,
                 prep,
                 xn_o, un_o, a_o, qu_o):
    has_lv = lv_r is not None
    nn = xv_r.shape[0]
    agg = agg_r[0, :nn, :64] + agg_r[1, :nn, :64]
    pre = (jnp.dot(xv_r[...], wxa[...], preferred_element_type=F32)
           + jnp.dot(agg, wagg[...], preferred_element_type=F32)
           + uu_r[...])
    if has_lv:
        pre = pre + jnp.dot(lv_r[...], wxl[...], preferred_element_type=F32)
    h = jnp.maximum(pre, 0.0)
    h = jnp.maximum(jnp.dot(h, nW2[...], preferred_element_type=F32) + nb2[...], 0.0)
    m = jnp.mean(h, axis=-1, keepdims=True)
    v = jnp.mean((h - m) * (h - m), axis=-1, keepdims=True)
    xn = (h - m) * lax.rsqrt(v + 1e-5) * ng[...] + nb[...]
    xn_o[...] = xn
    xmean = jnp.mean(xn, axis=0, keepdims=True)
    emean = esum_r[...] * (1.0 / esum_scale(agg_r.shape))
    gpre = (guu_r[...]
            + jnp.dot(xmean, gwx[...], preferred_element_type=F32)
            + jnp.dot(emean, gwe[...], preferred_element_type=F32))
    gh = jnp.maximum(gpre, 0.0)
    gh = jnp.maximum(jnp.dot(gh, gW2[...], preferred_element_type=F32) + gb2[...], 0.0)
    gm = jnp.mean(gh, axis=-1, keepdims=True)
    gv = jnp.mean((gh - gm) * (gh - gm), axis=-1, keepdims=True)
    un = (gh - gm) * lax.rsqrt(gv + 1e-5) * gg[...] + gb[...]
    un_o[...] = un
    if prep is not None:
        (wrx, wrl, wcx, wcl, wux, wul, b1e) = prep
        a = (jnp.dot(xv_r[...], wrx[...], preferred_element_type=F32)
             + jnp.dot(xn, wrl[...], preferred_element_type=F32))
        bb = (jnp.dot(xv_r[...], wcx[...], preferred_element_type=F32)
              + jnp.dot(xn, wcl[...], preferred_element_type=F32))
        a_o[...] = jnp.concatenate([a, bb], axis=1)
        qu_o[...] = (jnp.dot(ug_r[...], wux[...], preferred_element_type=F32)
                     + jnp.dot(un, wul[...], preferred_element_type=F32)
                     + b1e[...])


_E_TOTAL = [320000]


def esum_scale(shape):
    return float(_E_TOTAL[0])


def node_core(xv, lv, aggp, uu, esum, ug, lu, wxa, wxl, wagg, pnode, gu_pre,
              gwx, gwe, pglob, prep_ws):
    N = xv.shape[0]
    (_, _), (nW2, nb2) = pnode["lins"]
    (_, _), (gW2, gb2) = pglob["lins"]
    outs = [jax.ShapeDtypeStruct((N, 64), F32), jax.ShapeDtypeStruct((1, 32), F32)]
    if prep_ws is not None:
        outs += [jax.ShapeDtypeStruct((N, 128), F32),
                 jax.ShapeDtypeStruct((1, 64), F32)]

    have_lv = lv is not None

    def body(*refs):
        nin = 7 + 3 + 4 + 7 + (7 if prep_ws is not None else 0)
        idx = 0
        xv_r = refs[idx]; idx += 1
        if have_lv:
            lv_r = refs[idx]; idx += 1
        else:
            lv_r = None
        agg_r = refs[idx]; idx += 1
        uu_r = refs[idx]; idx += 1
        esum_r = refs[idx]; idx += 1
        ug_r = refs[idx]; idx += 1
        if prep_ws is not None and have_lv:
            pass
        wxa_r = refs[idx]; idx += 1
        wxl_r = None
        if have_lv:
            wxl_r = refs[idx]; idx += 1
        wagg_r = refs[idx]; idx += 1
        nW2_r = refs[idx]; idx += 1
        nb2_r = refs[idx]; idx += 1
        ng_r = refs[idx]; idx += 1
        nb_r = refs[idx]; idx += 1
        guu_r = refs[idx]; idx += 1
        gwx_r = refs[idx]; idx += 1
        gwe_r = refs[idx]; idx += 1
        gW2_r = refs[idx]; idx += 1
        gb2_r = refs[idx]; idx += 1
        gg_r = refs[idx]; idx += 1
        gb_r = refs[idx]; idx += 1
        prep_r = None
        if prep_ws is not None:
            prep_r = refs[idx:idx + 7]
            idx += 7
        out_rs = refs[idx:]
        xn_o, un_o = out_rs[0], out_rs[1]
        a_o = qu_o = None
        if prep_ws is not None:
            a_o, qu_o = out_rs[2], out_rs[3]
        _k_node_core(xv_r, lv_r, agg_r, uu_r, esum_r, ug_r, None,
                     wxa_r, wxl_r, wagg_r, nW2_r, nb2_r, ng_r, nb_r,
                     guu_r, gwx_r, gwe_r, gW2_r, gb2_r, gg_r, gb_r,
                     prep_r, xn_o, un_o, a_o, qu_o)

    ins = [xv]
    if have_lv:
        ins.append(lv)
    ins += [aggp, uu, esum, ug, wxa]
    if have_lv:
        ins.append(wxl)
    ins += [wagg, nW2, _row(nb2), _row(pnode["g"]), _row(pnode["b"]),
            gu_pre, gwx, gwe, gW2, _row(gb2), _row(pglob["g"]), _row(pglob["b"])]
    if prep_ws is not None:
        ins += list(prep_ws)
    return pl.pallas_call(body, out_shape=tuple(outs))(*ins)


def _k_dec_node(lv_r, lu_r, dW1, db1, dW2, db2, dg, db, wv, bv,
                uW1, ub1, uW2, ub2, ugg, ubb, wu, bu, ov_o, ou_o):
    d = _mlp_ln(lv_r[...], dW1[...], db1[...], dW2[...], db2[...], dg[...], db[...])
    ov_o[...] = jnp.dot(d, wv[...], preferred_element_type=F32) + bv[...]
    du = _mlp_ln(lu_r[...], uW1[...], ub1[...], uW2[...], ub2[...], ugg[...], ubb[...])
    ou_o[...] = jnp.dot(du, wu[...], preferred_element_type=F32) + bu[...]


def dec_node_glob(lv, lu, pnode, wv, bv, pglob, wu, bu):
    N = lv.shape[0]
    return pl.pallas_call(
        _k_dec_node,
        out_shape=(jax.ShapeDtypeStruct((N, 128), F32),
                   jax.ShapeDtypeStruct((1, 16), F32)),
    )(lv, lu, *_mlp_args(pnode), wv, _row(bv), *_mlp_args(pglob), wu, _row(bu))


def sc_gather(T, Q, row, col):
    E = Q.shape[0]
    nchunk = E // (_NW * _CH)
    per_w = nchunk * _CH
    mesh = plsc.VectorSubcoreMesh(core_axis_name="c", subcore_axis_name="s")

    @functools.partial(
        pl.kernel,
        out_type=jax.ShapeDtypeStruct((E, 64), F32),
        mesh=mesh,
        scratch_types=[
            pltpu.VMEM((_CH,), jnp.int32),
            pltpu.VMEM((_CH,), jnp.int32),
            pltpu.VMEM((_CH, 128), F32),
            pltpu.VMEM((_CH, 128), F32),
            pltpu.VMEM((_CH, 64), F32),
            pltpu.SemaphoreType.DMA,
            pltpu.SemaphoreType.DMA,
        ],
    )
    def k(t_hbm, q_hbm, row_hbm, col_hbm, out_hbm,
          idxr, idxc, bufa, bufb, bufq, sema, semb):
        wid = lax.axis_index("s") * _NC + lax.axis_index("c")
        base = wid * per_w

        def chunk(ci, carry):
            cb = base + ci * _CH
            pltpu.sync_copy(row_hbm.at[pl.ds(cb, _CH)], idxr)
            pltpu.sync_copy(col_hbm.at[pl.ds(cb, _CH)], idxc)
            cpa = pltpu.async_copy(t_hbm.at[idxr], bufa, sema)
            cpb = pltpu.async_copy(t_hbm.at[idxc], bufb, semb)
            pltpu.sync_copy(q_hbm.at[pl.ds(cb, _CH)], bufq)
            cpa.wait()
            cpb.wait()

            def rowbody(i, c2):
                for j in range(4):
                    s = pl.ds(j * 16, 16)
                    s2 = pl.ds(64 + j * 16, 16)
                    bufq[i, s] = jnp.maximum(bufa[i, s] + bufb[i, s2] + bufq[i, s],
                                             0.0)
                return c2

            lax.fori_loop(0, _CH, rowbody, 0, unroll=4)
            pltpu.sync_copy(bufq, out_hbm.at[pl.ds(cb, _CH)])
            return carry

        lax.fori_loop(0, nchunk, chunk, 0)

    return k(T, Q, row, col)


def sc_scatter(e_new, row, N):
    E = e_new.shape[0]
    nchunk = E // (_NW * _CH)
    per_w = nchunk * _CH
    Np = ((N + _NS * 128 - 1) // (_NS * 128)) * (_NS * 128)
    rows_sub = Np // _NS
    zrows = 128
    nz = rows_sub // zrows
    mesh = plsc.VectorSubcoreMesh(core_axis_name="c", subcore_axis_name="s")

    @functools.partial(
        pl.kernel,
        out_type=jax.ShapeDtypeStruct((_NC, Np, 128), F32),
        mesh=mesh,
        scratch_types=[
            pltpu.VMEM((1, _CH), jnp.int32),
            pltpu.VMEM((_CH, 128), F32),
            pltpu.VMEM((zrows, 128), F32),
            pltpu.VMEM_SHARED((Np, 128), F32),
        ],
    )
    def k(e_hbm, row2_hbm, out_hbm, idx, buf, zbuf, acc):
        cid = lax.axis_index("c")
        sid = lax.axis_index("s")
        wid = sid * _NC + cid

        def zb(i, c):
            for j in range(8):
                zbuf[i, pl.ds(j * 16, 16)] = jnp.zeros((16,), F32)
            return c

        lax.fori_loop(0, zrows, zb, 0, unroll=4)
        for r in range(nz):
            pltpu.sync_copy(zbuf, acc.at[pl.ds(sid * rows_sub + r * zrows, zrows)])
        plsc.subcore_barrier()

        def chunk(ci, c):
            ck = wid * nchunk + ci
            pltpu.sync_copy(row2_hbm.at[pl.ds(ck, 1)], idx)
            pltpu.sync_copy(e_hbm.at[pl.ds(ck * _CH, _CH)], buf)
            pltpu.sync_copy(buf, acc.at[idx.at[0]], add=True)
            return c

        lax.fori_loop(0, nchunk, chunk, 0)
        plsc.subcore_barrier()
        for r in range(nz):
            sl = pl.ds(sid * rows_sub + r * zrows, zrows)
            pltpu.sync_copy(acc.at[sl], out_hbm.at[cid, sl])

    return k(e_new, row.reshape(-1, _CH))


def kernel(x, edge_index, edge_attr, u, embed, params):
    row = edge_index[0]
    col = edge_index[1]
    N = x.shape[0]
    E = edge_attr.shape[0]
    _E_TOTAL[0] = E
    p = params

    W1e, b1e = p["core_edge"]["lins"][0]
    b1e = _row(b1e)
    wrx, wrl = W1e[0:64], W1e[64:128]
    wcx, wcl = W1e[128:192], W1e[192:256]
    wee, wel = W1e[256:320], W1e[320:384]
    wux, wul = W1e[384:416], W1e[416:448]

    W1n, b1n = p["core_node"]["lins"][0]
    b1n = _row(b1n)
    wxa, wxl = W1n[0:64], W1n[64:128]
    wagg = W1n[128:192]
    wnux, wnul = W1n[192:224], W1n[224:256]

    W1g, b1g = p["core_glob"]["lins"][0]
    b1g = _row(b1g)
    wgux, wgul = W1g[0:32], W1g[32:64]
    gwx, gwe = W1g[64:128], W1g[128:192]

    xv, ug, T1, qu1 = node_enc(x, embed, u, p, wrx, wcx, wux, b1e)

    e, Q1 = edge_enc(edge_attr, p, wee, qu1)

    H1 = sc_gather(T1, Q1, row, col)
    le, esum1 = edge_b(H1, p["core_edge"])
    aggp1 = sc_scatter(le, row, N)
    uu0 = jnp.dot(ug, wnux) + b1n
    guu0 = jnp.dot(ug, wgux) + b1g
    lv, lu, T2, qu2 = node_core(
        xv, None, aggp1, uu0, esum1, ug, None, wxa, None, wagg,
        p["core_node"], guu0, gwx, gwe, p["core_glob"],
        (wrx, wrl, wcx, wcl, wux, wul, b1e))

    Q2 = edge_q2(e, le, wee, wel, qu2)
    H2 = sc_gather(T2, Q2, row, col)
    le2, oe, esum2 = edge_b_dec(H2, p["core_edge"], p["dec_edge"],
                                p["eout"][0], p["eout"][1])
    aggp2 = sc_scatter(le2, row, N)
    uu1 = jnp.dot(ug, wnux) + jnp.dot(lu, wnul) + b1n
    guu1 = jnp.dot(ug, wgux) + jnp.dot(lu, wgul) + b1g
    lv2, lu2 = node_core(
        xv, lv, aggp2, uu1, esum2, ug, lu, wxa, wxl, wagg,
        p["core_node"], guu1, gwx, gwe, p["core_glob"], None)

    ov, ou = dec_node_glob(lv2, lu2, p["dec_node"], p["vout"][0], p["vout"][1],
                           p["dec_glob"], p["uout"][0], p["uout"][1])
    return (ov, oe, ou)


# speedup vs baseline: 3.2971x; 1.3653x over previous
import functools

import jax
import jax.numpy as jnp
from jax import lax
from jax.experimental import pallas as pl
from jax.experimental.pallas import tpu as pltpu
from jax.experimental.pallas import tpu_sc as plsc

F32 = jnp.float32

_NC = 2
_NS = 16
_NW = _NC * _NS
_CH = 80

_RB = 3200


def _mlp_ln(h, W1, b1, W2, b2, g, b):
    h = jnp.maximum(jnp.dot(h, W1, preferred_element_type=F32) + b1, 0.0)
    h = jnp.maximum(jnp.dot(h, W2, preferred_element_type=F32) + b2, 0.0)
    m = jnp.mean(h, axis=-1, keepdims=True)
    v = jnp.mean((h - m) * (h - m), axis=-1, keepdims=True)
    return (h - m) * lax.rsqrt(v + 1e-5) * g + b


def _row(a):
    return a.reshape(1, -1)


def _mlp_args(p):
    (W1, b1), (W2, b2) = p["lins"]
    return (W1, _row(b1), W2, _row(b2), _row(p["g"]), _row(p["b"]))


def _k_node_enc(x_r, emb_r, u_r,
                nW1, nb1, nW2, nb2, ng, nb,
                gW1, gb1, gW2, gb2, gg, gb,
                wra, wca, wua, b1e,
                xv_o, ug_o, a_o, qu_o):
    xv = _mlp_ln(x_r[...] + emb_r[...], nW1[...], nb1[...], nW2[...], nb2[...],
                 ng[...], nb[...])
    ug = _mlp_ln(u_r[...], gW1[...], gb1[...], gW2[...], gb2[...], gg[...], gb[...])
    xv_o[...] = xv
    ug_o[...] = ug
    a_o[...] = jnp.concatenate(
        [jnp.dot(xv, wra[...], preferred_element_type=F32),
         jnp.dot(xv, wca[...], preferred_element_type=F32)], axis=1)
    qu_o[...] = jnp.dot(ug, wua[...], preferred_element_type=F32) + b1e[...]


def node_enc(x, embed, u, p, wra, wca, wua, b1e):
    N = x.shape[0]
    out_shape = (
        jax.ShapeDtypeStruct((N, 64), F32),
        jax.ShapeDtypeStruct((1, 32), F32),
        jax.ShapeDtypeStruct((N, 128), F32),
        jax.ShapeDtypeStruct((1, 64), F32),
    )
    return pl.pallas_call(_k_node_enc, out_shape=out_shape)(
        x, embed, u, *_mlp_args(p["enc_node"]), *_mlp_args(p["enc_glob"]),
        wra, wca, wua, b1e)


def _k_edge_enc(ea_r, W1, b1, W2, b2, g, b, we, qu, e_o, q_o):
    e = _mlp_ln(ea_r[...], W1[...], b1[...], W2[...], b2[...], g[...], b[...])
    e_o[...] = e
    q_o[...] = jnp.dot(e, we[...], preferred_element_type=F32) + qu[...]


def edge_enc(edge_attr, p, we, qu):
    E = edge_attr.shape[0]
    grid = E // _RB
    full = lambda a: pl.BlockSpec(a.shape, lambda i: (0,) * a.ndim)
    args = (*_mlp_args(p["enc_edge"]), we, qu)
    return pl.pallas_call(
        _k_edge_enc,
        grid=(grid,),
        in_specs=[pl.BlockSpec((_RB, 16), lambda i: (i, 0))] + [full(a) for a in args],
        out_specs=(pl.BlockSpec((_RB, 64), lambda i: (i, 0)),
                   pl.BlockSpec((_RB, 64), lambda i: (i, 0))),
        out_shape=(jax.ShapeDtypeStruct((E, 64), F32),
                   jax.ShapeDtypeStruct((E, 64), F32)),
    )(edge_attr, *args)


def _k_edge_q2(e_r, le_r, we, wl, qu, q_o):
    q_o[...] = (jnp.dot(e_r[...], we[...], preferred_element_type=F32)
                + jnp.dot(le_r[...][:, :64], wl[...], preferred_element_type=F32)
                + qu[...])


def edge_q2(e, le, we, wl, qu):
    E = e.shape[0]
    grid = E // _RB
    full = lambda a: pl.BlockSpec(a.shape, lambda i: (0,) * a.ndim)
    eb = pl.BlockSpec((_RB, 64), lambda i: (i, 0))
    eb2 = pl.BlockSpec((_RB, 128), lambda i: (i, 0))
    return pl.pallas_call(
        _k_edge_q2,
        grid=(grid,),
        in_specs=[eb, eb2, full(we), full(wl), full(qu)],
        out_specs=eb,
        out_shape=jax.ShapeDtypeStruct((E, 64), F32),
    )(e, le, we, wl, qu)


def _k_edge_b(h_r, W2, b2, g, b, en_o, es_o):
    i = pl.program_id(0)
    h = jnp.maximum(jnp.dot(h_r[...], W2[...], preferred_element_type=F32) + b2[...], 0.0)
    m = jnp.mean(h, axis=-1, keepdims=True)
    v = jnp.mean((h - m) * (h - m), axis=-1, keepdims=True)
    en = (h - m) * lax.rsqrt(v + 1e-5) * g[...] + b[...]
    en_o[...] = jnp.concatenate([en, jnp.zeros_like(en)], axis=1)

    @pl.when(i == 0)
    def _():
        es_o[...] = jnp.zeros_like(es_o)

    es_o[...] += jnp.sum(en, axis=0, keepdims=True)


def edge_b(H, p):
    E = H.shape[0]
    grid = E // _RB
    (_, _), (W2, b2) = p["lins"]
    args = (W2, _row(b2), _row(p["g"]), _row(p["b"]))
    full = lambda a: pl.BlockSpec(a.shape, lambda i: (0,) * a.ndim)
    eb = pl.BlockSpec((_RB, 64), lambda i: (i, 0))
    eb2 = pl.BlockSpec((_RB, 128), lambda i: (i, 0))
    return pl.pallas_call(
        _k_edge_b,
        grid=(grid,),
        in_specs=[eb] + [full(a) for a in args],
        out_specs=(eb2, pl.BlockSpec((1, 64), lambda i: (0, 0))),
        out_shape=(jax.ShapeDtypeStruct((E, 128), F32),
                   jax.ShapeDtypeStruct((1, 64), F32)),
    )(H, *args)


def _k_edge_b_dec(h_r, W2, b2, g, b, dW1, db1, dW2, db2, dg, db, wo, bo,
                  en_o, oe_o, es_o):
    i = pl.program_id(0)
    h = jnp.maximum(jnp.dot(h_r[...], W2[...], preferred_element_type=F32) + b2[...], 0.0)
    m = jnp.mean(h, axis=-1, keepdims=True)
    v = jnp.mean((h - m) * (h - m), axis=-1, keepdims=True)
    en = (h - m) * lax.rsqrt(v + 1e-5) * g[...] + b[...]
    en_o[...] = jnp.concatenate([en, jnp.zeros_like(en)], axis=1)
    d = _mlp_ln(en, dW1[...], db1[...], dW2[...], db2[...], dg[...], db[...])
    oe_o[...] = jnp.dot(d, wo[...], preferred_element_type=F32) + bo[...]

    @pl.when(i == 0)
    def _():
        es_o[...] = jnp.zeros_like(es_o)

    es_o[...] += jnp.sum(en, axis=0, keepdims=True)


def edge_b_dec(H, p, pdec, wo, bo):
    E = H.shape[0]
    grid = E // _RB
    (_, _), (W2, b2) = p["lins"]
    args = (W2, _row(b2), _row(p["g"]), _row(p["b"]), *_mlp_args(pdec), wo, _row(bo))
    full = lambda a: pl.BlockSpec(a.shape, lambda i: (0,) * a.ndim)
    eb = pl.BlockSpec((_RB, 64), lambda i: (i, 0))
    eb2 = pl.BlockSpec((_RB, 128), lambda i: (i, 0))
    return pl.pallas_call(
        _k_edge_b_dec,
        grid=(grid,),
        in_specs=[eb] + [full(a) for a in args],
        out_specs=(eb2, pl.BlockSpec((_RB, 16), lambda i: (i, 0)),
                   pl.BlockSpec((1, 64), lambda i: (0, 0))),
        out_shape=(jax.ShapeDtypeStruct((E, 128), F32),
                   jax.ShapeDtypeStruct((E, 16), F32),
                   jax.ShapeDtypeStruct((1, 64), F32)),
    )(H, *args)


def _k_node_core(xv_r, lv_r, agg_r, uu_r, esum_r, ug_r, lu_r,
                 wxa, wxl, wagg,
                 nW2, nb2, ng, nb,
                 guu_r, gwx, gwe, gW2, gb2, gg, gb,
                 prep,
                 xn_o, un_o, a_o, qu_o):
    has_lv = lv_r is not None
    nn = xv_r.shape[0]
    agg = agg_r[0, :nn, :64] + agg_r[1, :nn, :64]
    pre = (jnp.dot(xv_r[...], wxa[...], preferred_element_type=F32)
           + jnp.dot(agg, wagg[...], preferred_element_type=F32)
           + uu_r[...])
    if has_lv:
        pre = pre + jnp.dot(lv_r[...], wxl[...], preferred_element_type=F32)
    h = jnp.maximum(pre, 0.0)
    h = jnp.maximum(jnp.dot(h, nW2[...], preferred_element_type=F32) + nb2[...], 0.0)
    m = jnp.mean(h, axis=-1, keepdims=True)
    v = jnp.mean((h - m) * (h - m), axis=-1, keepdims=True)
    xn = (h - m) * lax.rsqrt(v + 1e-5) * ng[...] + nb[...]
    xn_o[...] = xn
    xmean = jnp.mean(xn, axis=0, keepdims=True)
    emean = esum_r[...] * (1.0 / esum_scale(agg_r.shape))
    gpre = (guu_r[...]
            + jnp.dot(xmean, gwx[...], preferred_element_type=F32)
            + jnp.dot(emean, gwe[...], preferred_element_type=F32))
    gh = jnp.maximum(gpre, 0.0)
    gh = jnp.maximum(jnp.dot(gh, gW2[...], preferred_element_type=F32) + gb2[...], 0.0)
    gm = jnp.mean(gh, axis=-1, keepdims=True)
    gv = jnp.mean((gh - gm) * (gh - gm), axis=-1, keepdims=True)
    un = (gh - gm) * lax.rsqrt(gv + 1e-5) * gg[...] + gb[...]
    un_o[...] = un
    if prep is not None:
        (wrx, wrl, wcx, wcl, wux, wul, b1e) = prep
        a = (jnp.dot(xv_r[...], wrx[...], preferred_element_type=F32)
             + jnp.dot(xn, wrl[...], preferred_element_type=F32))
        bb = (jnp.dot(xv_r[...], wcx[...], preferred_element_type=F32)
              + jnp.dot(xn, wcl[...], preferred_element_type=F32))
        a_o[...] = jnp.concatenate([a, bb], axis=1)
        qu_o[...] = (jnp.dot(ug_r[...], wux[...], preferred_element_type=F32)
                     + jnp.dot(un, wul[...], preferred_element_type=F32)
                     + b1e[...])


_E_TOTAL = [320000]


def esum_scale(shape):
    return float(_E_TOTAL[0])


def node_core(xv, lv, aggp, uu, esum, ug, lu, wxa, wxl, wagg, pnode, gu_pre,
              gwx, gwe, pglob, prep_ws):
    N = xv.shape[0]
    (_, _), (nW2, nb2) = pnode["lins"]
    (_, _), (gW2, gb2) = pglob["lins"]
    outs = [jax.ShapeDtypeStruct((N, 64), F32), jax.ShapeDtypeStruct((1, 32), F32)]
    if prep_ws is not None:
        outs += [jax.ShapeDtypeStruct((N, 128), F32),
                 jax.ShapeDtypeStruct((1, 64), F32)]

    have_lv = lv is not None

    def body(*refs):
        nin = 7 + 3 + 4 + 7 + (7 if prep_ws is not None else 0)
        idx = 0
        xv_r = refs[idx]; idx += 1
        if have_lv:
            lv_r = refs[idx]; idx += 1
        else:
            lv_r = None
        agg_r = refs[idx]; idx += 1
        uu_r = refs[idx]; idx += 1
        esum_r = refs[idx]; idx += 1
        ug_r = refs[idx]; idx += 1
        if prep_ws is not None and have_lv:
            pass
        wxa_r = refs[idx]; idx += 1
        wxl_r = None
        if have_lv:
            wxl_r = refs[idx]; idx += 1
        wagg_r = refs[idx]; idx += 1
        nW2_r = refs[idx]; idx += 1
        nb2_r = refs[idx]; idx += 1
        ng_r = refs[idx]; idx += 1
        nb_r = refs[idx]; idx += 1
        guu_r = refs[idx]; idx += 1
        gwx_r = refs[idx]; idx += 1
        gwe_r = refs[idx]; idx += 1
        gW2_r = refs[idx]; idx += 1
        gb2_r = refs[idx]; idx += 1
        gg_r = refs[idx]; idx += 1
        gb_r = refs[idx]; idx += 1
        prep_r = None
        if prep_ws is not None:
            prep_r = refs[idx:idx + 7]
            idx += 7
        out_rs = refs[idx:]
        xn_o, un_o = out_rs[0], out_rs[1]
        a_o = qu_o = None
        if prep_ws is not None:
            a_o, qu_o = out_rs[2], out_rs[3]
        _k_node_core(xv_r, lv_r, agg_r, uu_r, esum_r, ug_r, None,
                     wxa_r, wxl_r, wagg_r, nW2_r, nb2_r, ng_r, nb_r,
                     guu_r, gwx_r, gwe_r, gW2_r, gb2_r, gg_r, gb_r,
                     prep_r, xn_o, un_o, a_o, qu_o)

    ins = [xv]
    if have_lv:
        ins.append(lv)
    ins += [aggp, uu, esum, ug, wxa]
    if have_lv:
        ins.append(wxl)
    ins += [wagg, nW2, _row(nb2), _row(pnode["g"]), _row(pnode["b"]),
            gu_pre, gwx, gwe, gW2, _row(gb2), _row(pglob["g"]), _row(pglob["b"])]
    if prep_ws is not None:
        ins += list(prep_ws)
    return pl.pallas_call(body, out_shape=tuple(outs))(*ins)


def _k_dec_node(lv_r, lu_r, dW1, db1, dW2, db2, dg, db, wv, bv,
                uW1, ub1, uW2, ub2, ugg, ubb, wu, bu, ov_o, ou_o):
    d = _mlp_ln(lv_r[...], dW1[...], db1[...], dW2[...], db2[...], dg[...], db[...])
    ov_o[...] = jnp.dot(d, wv[...], preferred_element_type=F32) + bv[...]
    du = _mlp_ln(lu_r[...], uW1[...], ub1[...], uW2[...], ub2[...], ugg[...], ubb[...])
    ou_o[...] = jnp.dot(du, wu[...], preferred_element_type=F32) + bu[...]


def dec_node_glob(lv, lu, pnode, wv, bv, pglob, wu, bu):
    N = lv.shape[0]
    return pl.pallas_call(
        _k_dec_node,
        out_shape=(jax.ShapeDtypeStruct((N, 128), F32),
                   jax.ShapeDtypeStruct((1, 16), F32)),
    )(lv, lu, *_mlp_args(pnode), wv, _row(bv), *_mlp_args(pglob), wu, _row(bu))


def sc_gather(T, Q, row, col):
    E = Q.shape[0]
    nchunk = E // (_NW * _CH)
    per_w = nchunk * _CH
    npair = (nchunk - 1) // 2
    mesh = plsc.VectorSubcoreMesh(core_axis_name="c", subcore_axis_name="s")

    @functools.partial(
        pl.kernel,
        out_type=jax.ShapeDtypeStruct((E, 64), F32),
        mesh=mesh,
        scratch_types=[
            pltpu.VMEM((nchunk, _CH), jnp.int32),
            pltpu.VMEM((nchunk, _CH), jnp.int32),
            pltpu.VMEM((2, _CH, 128), F32),
            pltpu.VMEM((2, _CH, 128), F32),
            pltpu.VMEM((2, _CH, 64), F32),
            pltpu.VMEM((2, _CH, 64), F32),
            [pltpu.SemaphoreType.DMA] * 2,
            [pltpu.SemaphoreType.DMA] * 2,
            [pltpu.SemaphoreType.DMA] * 2,
            [pltpu.SemaphoreType.DMA] * 2,
        ],
    )
    def k(t_hbm, q_hbm, row3_hbm, col3_hbm, out_hbm,
          idxr, idxc, bufa, bufb, bufq, bufh, sema, semb, semq, semw):
        wid = lax.axis_index("s") * _NC + lax.axis_index("c")
        base = wid * per_w

        pltpu.sync_copy(row3_hbm.at[wid], idxr)
        pltpu.sync_copy(col3_hbm.at[wid], idxc)

        def issue(ci, b):
            pltpu.async_copy(t_hbm.at[idxr.at[ci]], bufa.at[b], sema[b])
            pltpu.async_copy(t_hbm.at[idxc.at[ci]], bufb.at[b], semb[b])
            pltpu.async_copy(q_hbm.at[pl.ds(base + ci * _CH, _CH)], bufq.at[b],
                             semq[b])

        def halfstep(ci, b, issue_next):
            if issue_next:
                issue(ci + 1, 1 - b)
            pltpu.make_async_copy(t_hbm.at[idxr.at[ci]], bufa.at[b], sema[b]).wait()
            pltpu.make_async_copy(t_hbm.at[idxc.at[ci]], bufb.at[b], semb[b]).wait()
            pltpu.make_async_copy(q_hbm.at[pl.ds(base, _CH)], bufq.at[b],
                                  semq[b]).wait()

            @pl.when(ci >= 2)
            def _():
                pltpu.make_async_copy(bufh.at[b],
                                      out_hbm.at[pl.ds(base, _CH)], semw[b]).wait()

            ba, bb, bq, bh = bufa.at[b], bufb.at[b], bufq.at[b], bufh.at[b]

            def rowbody(r, c2):
                for j in range(4):
                    s = pl.ds(j * 16, 16)
                    s2 = pl.ds(64 + j * 16, 16)
                    bh[r, s] = jnp.maximum(ba[r, s] + bb[r, s2] + bq[r, s], 0.0)
                return c2

            lax.fori_loop(0, _CH, rowbody, 0, unroll=4)
            pltpu.async_copy(bufh.at[b], out_hbm.at[pl.ds(base + ci * _CH, _CH)],
                             semw[b])

        issue(0, 0)

        def pair(kk, c):
            halfstep(2 * kk, 0, True)
            halfstep(2 * kk + 1, 1, True)
            return c

        lax.fori_loop(0, npair, pair, 0)
        halfstep(nchunk - 1, 0, False)
        pltpu.make_async_copy(bufh.at[0], out_hbm.at[pl.ds(base, _CH)],
                              semw[0]).wait()
        pltpu.make_async_copy(bufh.at[1], out_hbm.at[pl.ds(base, _CH)],
                              semw[1]).wait()

    return k(T, Q, row.reshape(_NW, nchunk, _CH), col.reshape(_NW, nchunk, _CH))


def sc_scatter(e_new, row, N):
    E = e_new.shape[0]
    nchunk = E // (_NW * _CH)
    per_w = nchunk * _CH
    Np = ((N + _NS * 128 - 1) // (_NS * 128)) * (_NS * 128)
    rows_sub = Np // _NS
    zrows = 128
    nz = rows_sub // zrows
    mesh = plsc.VectorSubcoreMesh(core_axis_name="c", subcore_axis_name="s")

    @functools.partial(
        pl.kernel,
        out_type=jax.ShapeDtypeStruct((_NC, Np, 128), F32),
        mesh=mesh,
        scratch_types=[
            pltpu.VMEM((1, _CH), jnp.int32),
            pltpu.VMEM((_CH, 128), F32),
            pltpu.VMEM((zrows, 128), F32),
            pltpu.VMEM_SHARED((Np, 128), F32),
        ],
    )
    def k(e_hbm, row2_hbm, out_hbm, idx, buf, zbuf, acc):
        cid = lax.axis_index("c")
        sid = lax.axis_index("s")
        wid = sid * _NC + cid

        def zb(i, c):
            for j in range(8):
                zbuf[i, pl.ds(j * 16, 16)] = jnp.zeros((16,), F32)
            return c

        lax.fori_loop(0, zrows, zb, 0, unroll=4)
        for r in range(nz):
            pltpu.sync_copy(zbuf, acc.at[pl.ds(sid * rows_sub + r * zrows, zrows)])
        plsc.subcore_barrier()

        def chunk(ci, c):
            ck = wid * nchunk + ci
            pltpu.sync_copy(row2_hbm.at[pl.ds(ck, 1)], idx)
            pltpu.sync_copy(e_hbm.at[pl.ds(ck * _CH, _CH)], buf)
            pltpu.sync_copy(buf, acc.at[idx.at[0]], add=True)
            return c

        lax.fori_loop(0, nchunk, chunk, 0)
        plsc.subcore_barrier()
        for r in range(nz):
            sl = pl.ds(sid * rows_sub + r * zrows, zrows)
            pltpu.sync_copy(acc.at[sl], out_hbm.at[cid, sl])

    return k(e_new, row.reshape(-1, _CH))


def kernel(x, edge_index, edge_attr, u, embed, params):
    row = edge_index[0]
    col = edge_index[1]
    N = x.shape[0]
    E = edge_attr.shape[0]
    _E_TOTAL[0] = E
    p = params

    W1e, b1e = p["core_edge"]["lins"][0]
    b1e = _row(b1e)
    wrx, wrl = W1e[0:64], W1e[64:128]
    wcx, wcl = W1e[128:192], W1e[192:256]
    wee, wel = W1e[256:320], W1e[320:384]
    wux, wul = W1e[384:416], W1e[416:448]

    W1n, b1n = p["core_node"]["lins"][0]
    b1n = _row(b1n)
    wxa, wxl = W1n[0:64], W1n[64:128]
    wagg = W1n[128:192]
    wnux, wnul = W1n[192:224], W1n[224:256]

    W1g, b1g = p["core_glob"]["lins"][0]
    b1g = _row(b1g)
    wgux, wgul = W1g[0:32], W1g[32:64]
    gwx, gwe = W1g[64:128], W1g[128:192]

    xv, ug, T1, qu1 = node_enc(x, embed, u, p, wrx, wcx, wux, b1e)

    e, Q1 = edge_enc(edge_attr, p, wee, qu1)

    H1 = sc_gather(T1, Q1, row, col)
    le, esum1 = edge_b(H1, p["core_edge"])
    aggp1 = sc_scatter(le, row, N)
    uu0 = jnp.dot(ug, wnux) + b1n
    guu0 = jnp.dot(ug, wgux) + b1g
    lv, lu, T2, qu2 = node_core(
        xv, None, aggp1, uu0, esum1, ug, None, wxa, None, wagg,
        p["core_node"], guu0, gwx, gwe, p["core_glob"],
        (wrx, wrl, wcx, wcl, wux, wul, b1e))

    Q2 = edge_q2(e, le, wee, wel, qu2)
    H2 = sc_gather(T2, Q2, row, col)
    le2, oe, esum2 = edge_b_dec(H2, p["core_edge"], p["dec_edge"],
                                p["eout"][0], p["eout"][1])
    aggp2 = sc_scatter(le2, row, N)
    uu1 = jnp.dot(ug, wnux) + jnp.dot(lu, wnul) + b1n
    guu1 = jnp.dot(ug, wgux) + jnp.dot(lu, wgul) + b1g
    lv2, lu2 = node_core(
        xv, lv, aggp2, uu1, esum2, ug, lu, wxa, wxl, wagg,
        p["core_node"], guu1, gwx, gwe, p["core_glob"], None)

    ov, ou = dec_node_glob(lv2, lu2, p["dec_node"], p["vout"][0], p["vout"][1],
                           p["dec_glob"], p["uout"][0], p["uout"][1])
    return (ov, oe, ou)
